# Initial kernel scaffold; baseline (speedup 1.0000x reference)
import jax
import jax.numpy as jnp
from jax.experimental import pallas as pl


def kernel(positions, pe_weight):
    raise NotImplementedError("write your pallas kernel here")


# speedup vs baseline: 2.1832x; 2.1832x over previous
import functools

import jax
import jax.numpy as jnp
from jax import lax
from jax.experimental import pallas as pl
from jax.experimental.pallas import tpu as pltpu
from jax.experimental.pallas import tpu_sc as plsc

MAX_LEN = 8192
D_MODEL = 1024

_info = plsc.get_sparse_core_info()
NC, NS = _info.num_cores, _info.num_subcores
NW = NC * NS

B_TOTAL = 4 * 8192
B_PER_W = B_TOTAL // NW
CHUNK = 64
NCHUNK = B_PER_W // CHUNK


@functools.partial(
    pl.kernel,
    mesh=plsc.VectorSubcoreMesh(core_axis_name="c", subcore_axis_name="s"),
    out_type=jax.ShapeDtypeStruct((B_TOTAL, D_MODEL), jnp.float32),
    scratch_types=[
        pltpu.VMEM((NCHUNK, CHUNK), jnp.int32),
        pltpu.VMEM((CHUNK, D_MODEL), jnp.float32),
        pltpu.SemaphoreType.DMA,
    ],
)
def _emb_lookup(idx_hbm, table_hbm, out_hbm, idx_v, buf_v, sem):
    wid = lax.axis_index("s") * NC + lax.axis_index("c")
    base = wid * B_PER_W
    pltpu.sync_copy(idx_hbm.at[wid], idx_v)

    def chunk_body(j, carry):
        pltpu.async_copy(table_hbm.at[idx_v.at[j]], buf_v, sem).wait()
        pltpu.sync_copy(buf_v, out_hbm.at[pl.ds(base + j * CHUNK, CHUNK)])
        return carry

    lax.fori_loop(0, NCHUNK, chunk_body, 0)


def kernel(positions, pe_weight):
    idx = positions.reshape(NW, NCHUNK, CHUNK).astype(jnp.int32)
    out = _emb_lookup(idx, pe_weight)
    return out.reshape(positions.shape + (D_MODEL,))


# speedup vs baseline: 2.2976x; 1.0524x over previous
import functools

import jax
import jax.numpy as jnp
from jax import lax
from jax.experimental import pallas as pl
from jax.experimental.pallas import tpu as pltpu
from jax.experimental.pallas import tpu_sc as plsc

MAX_LEN = 8192
D_MODEL = 1024

_info = plsc.get_sparse_core_info()
NC, NS = _info.num_cores, _info.num_subcores
NW = NC * NS

B_TOTAL = 4 * 8192
B_PER_W = B_TOTAL // NW
CHUNK = 32
NCHUNK = B_PER_W // CHUNK
NBUF = 2


@functools.partial(
    pl.kernel,
    mesh=plsc.VectorSubcoreMesh(core_axis_name="c", subcore_axis_name="s"),
    out_type=jax.ShapeDtypeStruct((B_TOTAL, D_MODEL), jnp.float32),
    scratch_types=[
        pltpu.VMEM((NCHUNK, CHUNK), jnp.int32),
        pltpu.VMEM((NBUF, CHUNK, D_MODEL), jnp.float32),
        pltpu.SemaphoreType.DMA,
        pltpu.SemaphoreType.DMA,
    ],
)
def _emb_lookup(idx_hbm, table_hbm, out_hbm, idx_v, buf_v, gsem, ssem):
    wid = lax.axis_index("s") * NC + lax.axis_index("c")
    base = wid * B_PER_W
    pltpu.sync_copy(idx_hbm.at[wid], idx_v)

    def gather_start(j):
        pltpu.async_copy(table_hbm.at[idx_v.at[j]], buf_v.at[j % NBUF], gsem)

    def gather_wait():
        pltpu.make_async_copy(
            table_hbm.at[pl.ds(0, CHUNK)], buf_v.at[0], gsem
        ).wait()

    def scatter_start(j):
        pltpu.async_copy(
            buf_v.at[j % NBUF], out_hbm.at[pl.ds(base + j * CHUNK, CHUNK)], ssem
        )

    def scatter_wait():
        pltpu.make_async_copy(
            buf_v.at[0], out_hbm.at[pl.ds(base, CHUNK)], ssem
        ).wait()

    gather_start(0)
    gather_wait()
    scatter_start(0)
    gather_start(1)

    def steady(j, carry):
        gather_wait()
        scatter_start(j)
        scatter_wait()
        gather_start(j + 1)
        return carry

    lax.fori_loop(1, NCHUNK - 1, steady, 0)

    gather_wait()
    scatter_start(NCHUNK - 1)
    scatter_wait()
    scatter_wait()


def kernel(positions, pe_weight):
    idx = positions.reshape(NW, NCHUNK, CHUNK).astype(jnp.int32)
    out = _emb_lookup(idx, pe_weight)
    return out.reshape(positions.shape + (D_MODEL,))


# speedup vs baseline: 2.3964x; 1.0430x over previous
import functools

import jax
import jax.numpy as jnp
from jax import lax
from jax.experimental import pallas as pl
from jax.experimental.pallas import tpu as pltpu
from jax.experimental.pallas import tpu_sc as plsc

MAX_LEN = 8192
D_MODEL = 1024

_info = plsc.get_sparse_core_info()
NC, NS = _info.num_cores, _info.num_subcores
NW = NC * NS

B_TOTAL = 4 * 8192
B_PER_W = B_TOTAL // NW
CHUNK = 32
NCHUNK = B_PER_W // CHUNK
NBUF = 3


@functools.partial(
    pl.kernel,
    mesh=plsc.VectorSubcoreMesh(core_axis_name="c", subcore_axis_name="s"),
    out_type=jax.ShapeDtypeStruct((B_TOTAL, D_MODEL), jnp.float32),
    scratch_types=[
        pltpu.VMEM((NCHUNK, CHUNK), jnp.int32),
        pltpu.VMEM((NBUF, CHUNK, D_MODEL), jnp.float32),
        pltpu.SemaphoreType.DMA,
        pltpu.SemaphoreType.DMA,
    ],
)
def _emb_lookup(idx_hbm, table_hbm, out_hbm, idx_v, buf_v, gsem, ssem):
    wid = lax.axis_index("s") * NC + lax.axis_index("c")
    base = wid * B_PER_W
    pltpu.sync_copy(idx_hbm.at[wid], idx_v)

    def gather_start(j):
        pltpu.async_copy(table_hbm.at[idx_v.at[j]], buf_v.at[j % NBUF], gsem)

    def gather_wait():
        pltpu.make_async_copy(
            table_hbm.at[pl.ds(0, CHUNK)], buf_v.at[0], gsem
        ).wait()

    def scatter_start(j):
        pltpu.async_copy(
            buf_v.at[j % NBUF], out_hbm.at[pl.ds(base + j * CHUNK, CHUNK)], ssem
        )

    def scatter_wait():
        pltpu.make_async_copy(
            buf_v.at[0], out_hbm.at[pl.ds(base, CHUNK)], ssem
        ).wait()

    gather_start(0)
    gather_start(1)

    gather_wait()
    scatter_start(0)
    gather_start(2)

    def steady(j, carry):
        gather_wait()
        scatter_start(j)
        scatter_wait()
        gather_start(j + 2)
        return carry

    lax.fori_loop(1, NCHUNK - 2, steady, 0)

    gather_wait()
    scatter_start(NCHUNK - 2)
    gather_wait()
    scatter_start(NCHUNK - 1)
    scatter_wait()
    scatter_wait()
    scatter_wait()


def kernel(positions, pe_weight):
    idx = positions.reshape(NW, NCHUNK, CHUNK).astype(jnp.int32)
    out = _emb_lookup(idx, pe_weight)
    return out.reshape(positions.shape + (D_MODEL,))
